# Initial kernel scaffold; baseline (speedup 1.0000x reference)
import jax
import jax.numpy as jnp
from jax.experimental import pallas as pl


def kernel(features_1, segment_ids):
    raise NotImplementedError("write your pallas kernel here")


# speedup vs baseline: 1.2797x; 1.2797x over previous
import functools

import jax
import jax.numpy as jnp
from jax import lax
from jax.experimental import pallas as pl
from jax.experimental.pallas import tpu as pltpu
from jax.experimental.pallas import tpu_sc as plsc

N = 100000
D = 128
NUM_GRAPHS = 128
F = 3 * D
NC = 2
NS = 16
L = 16
T = 32
NSUB = N // T
SUB_PER_S = NSUB // NS
SUB_EXTRA = NSUB % NS
MAX_SUB = SUB_PER_S + 1
IDS_ROWS = 3128
CPS = F // NC
GRP = CPS // L


def _sc_body(f_hbm, ids_hbm, out_hbm,
             acc, cnt, ids_v, buf, ones_v, zbuf, czbuf, fbuf, cbuf):
    c = lax.axis_index("c")
    s = lax.axis_index("s")
    col0 = c * CPS

    zero = jnp.zeros((L,), jnp.float32)
    for r in range(8):
        for k in range(GRP):
            zbuf[r, pl.ds(k * L, L)] = zero
        czbuf[r, :] = zero
    pltpu.sync_copy(zbuf, acc.at[pl.ds(s * 8, 8)])
    pltpu.sync_copy(czbuf, cnt.at[pl.ds(s * 8, 8)])

    one = jnp.full((L,), 1.0, jnp.float32)
    for r in range(T):
        ones_v[r, :] = one

    plsc.subcore_barrier()

    base_sub = s * SUB_PER_S + jnp.minimum(s, SUB_EXTRA)
    nsub = SUB_PER_S + jnp.where(s < SUB_EXTRA, 1, 0)

    pltpu.sync_copy(ids_hbm.at[pl.ds(base_sub, MAX_SUB)], ids_v)

    def body(j, carry):
        sidx = base_sub + j
        pltpu.sync_copy(
            f_hbm.at[pl.ds(sidx * T, T), pl.ds(col0, CPS)], buf)
        pltpu.sync_copy(buf, acc.at[ids_v.at[j]], add=True)
        pltpu.sync_copy(ones_v, cnt.at[ids_v.at[j]], add=True)
        return carry

    lax.fori_loop(0, nsub, body, 0)

    plsc.subcore_barrier()

    pltpu.sync_copy(acc.at[pl.ds(s * 8, 8)], fbuf)
    pltpu.sync_copy(cnt.at[pl.ds(s * 8, 8)], cbuf)
    for r in range(8):
        cv = cbuf[r, :]
        rec = jnp.full((L,), 1.0, jnp.float32) / jnp.maximum(
            cv, jnp.full((L,), 1.0, jnp.float32))
        for k in range(GRP):
            fbuf[r, pl.ds(k * L, L)] = fbuf[r, pl.ds(k * L, L)] * rec
    pltpu.sync_copy(fbuf, out_hbm.at[pl.ds(s * 8, 8), pl.ds(col0, CPS)])


@jax.jit
def _gavg_pool(f, ids2d):
    mesh = plsc.VectorSubcoreMesh(core_axis_name="c", subcore_axis_name="s")
    return pl.kernel(
        _sc_body,
        out_type=jax.ShapeDtypeStruct((NUM_GRAPHS, F), jnp.float32),
        mesh=mesh,
        compiler_params=pltpu.CompilerParams(use_tc_tiling_on_sc=False),
        scratch_types=[
            pltpu.VMEM_SHARED((NUM_GRAPHS, CPS), jnp.float32),
            pltpu.VMEM_SHARED((NUM_GRAPHS, L), jnp.float32),
            pltpu.VMEM((MAX_SUB, T), jnp.int32),
            pltpu.VMEM((T, CPS), jnp.float32),
            pltpu.VMEM((T, L), jnp.float32),
            pltpu.VMEM((8, CPS), jnp.float32),
            pltpu.VMEM((8, L), jnp.float32),
            pltpu.VMEM((8, CPS), jnp.float32),
            pltpu.VMEM((8, L), jnp.float32),
        ],
    )(f, ids2d)


def kernel(features_1, segment_ids):
    f = features_1.reshape(N, F)
    ids = segment_ids.astype(jnp.int32)
    ids = jnp.pad(ids, (0, IDS_ROWS * T - N),
                  constant_values=NUM_GRAPHS - 1).reshape(IDS_ROWS, T)
    raw = _gavg_pool(f, ids)
    return raw.reshape(NUM_GRAPHS, D, 3).transpose(0, 2, 1).reshape(
        NUM_GRAPHS, F)


# speedup vs baseline: 1.4780x; 1.1549x over previous
import jax
import jax.numpy as jnp
from jax import lax
from jax.experimental import pallas as pl
from jax.experimental.pallas import tpu as pltpu
from jax.experimental.pallas import tpu_sc as plsc

N = 100000
D = 128
NUM_GRAPHS = 128
F = 3 * D
NC = 2
NS = 16
L = 16
T = 125
NSUB = N // T
SUB_PER_S = NSUB // NS
ROWS_PER_S = N // NS
CPS = F // NC
GRP = CPS // L


def _sc_body(f_hbm, ids2d_hbm, out_hbm,
             acc, cnt, ids_v, buf_a, buf_b,
             zbuf, czbuf, cloc, identr, fbuf, cbuf, sem_g, sem_s):
    c = lax.axis_index("c")
    s = lax.axis_index("s")
    col0 = c * CPS
    row0 = s * ROWS_PER_S

    zero = jnp.zeros((L,), jnp.float32)
    for r in range(8):
        for k in range(GRP):
            zbuf[r, pl.ds(k * L, L)] = zero
        czbuf[r, :] = zero
    pltpu.sync_copy(zbuf, acc.at[pl.ds(s * 8, 8)])
    pltpu.sync_copy(czbuf, cnt.at[pl.ds(s * 8, 8)])

    lanes = lax.iota(jnp.int32, L)
    ones = jnp.full((L,), 1.0, jnp.float32)
    for r in range(NUM_GRAPHS):
        cloc[r, :] = zero
    for k in range(NUM_GRAPHS // L):
        identr[0, pl.ds(k * L, L)] = lanes + jnp.full((L,), k * L, jnp.int32)
    pltpu.sync_copy(ids2d_hbm.at[pl.ds(s * SUB_PER_S, SUB_PER_S)], ids_v)
    tail_msk = lanes >= jnp.full((L,), 3, jnp.int32)
    for j in range(SUB_PER_S):
        for k in range(T // L):
            idv = ids_v[j, pl.ds(k * L, L)]
            plsc.addupdate_scatter(cloc, [idv, lanes], ones)
        idv = ids_v[j, pl.ds(T - L, L)]
        plsc.addupdate_scatter(cloc, [idv, lanes], ones, mask=tail_msk)


    bufs = (buf_a, buf_b)

    def start_gather(j):
        return pltpu.async_copy(
            f_hbm.at[pl.ds(row0 + j * T, T), pl.ds(col0, CPS)],
            bufs[j % 2], sem_g)

    def start_scatter(j):
        return pltpu.async_copy(
            bufs[j % 2], acc.at[ids_v.at[j]], sem_s, add=True)

    gd = start_gather(0)
    sd_prev = None
    for j in range(SUB_PER_S):
        gd.wait()
        if sd_prev is not None:
            sd_prev.wait()
        if j + 1 < SUB_PER_S:
            gd = start_gather(j + 1)
        sd_prev = start_scatter(j)
    sd_prev.wait()

    pltpu.sync_copy(cloc, cnt.at[identr.at[0]], add=True)

    plsc.subcore_barrier()

    pltpu.sync_copy(acc.at[pl.ds(s * 8, 8)], fbuf)
    pltpu.sync_copy(cnt.at[pl.ds(s * 8, 8)], cbuf)
    for r in range(8):
        total = jnp.sum(cbuf[r, :])
        rec = jnp.full((L,), 1.0, jnp.float32) / jnp.maximum(
            jnp.full((L,), total, jnp.float32),
            jnp.full((L,), 1.0, jnp.float32))
        for k in range(GRP):
            fbuf[r, pl.ds(k * L, L)] = fbuf[r, pl.ds(k * L, L)] * rec
    pltpu.sync_copy(fbuf, out_hbm.at[pl.ds(s * 8, 8), pl.ds(col0, CPS)])


@jax.jit
def _gavg_pool(f, ids2d):
    mesh = plsc.VectorSubcoreMesh(core_axis_name="c", subcore_axis_name="s")
    return pl.kernel(
        _sc_body,
        out_type=jax.ShapeDtypeStruct((NUM_GRAPHS, F), jnp.float32),
        mesh=mesh,
        compiler_params=pltpu.CompilerParams(
            use_tc_tiling_on_sc=False, needs_layout_passes=False),
        scratch_types=[
            pltpu.VMEM_SHARED((NUM_GRAPHS, CPS), jnp.float32),
            pltpu.VMEM_SHARED((NUM_GRAPHS, L), jnp.float32),
            pltpu.VMEM((SUB_PER_S, T), jnp.int32),
            pltpu.VMEM((T, CPS), jnp.float32),
            pltpu.VMEM((T, CPS), jnp.float32),
            pltpu.VMEM((8, CPS), jnp.float32),
            pltpu.VMEM((8, L), jnp.float32),
            pltpu.VMEM((NUM_GRAPHS, L), jnp.float32),
            pltpu.VMEM((1, NUM_GRAPHS), jnp.int32),
            pltpu.VMEM((8, CPS), jnp.float32),
            pltpu.VMEM((8, L), jnp.float32),
            pltpu.SemaphoreType.DMA,
            pltpu.SemaphoreType.DMA,
        ],
    )(f, ids2d)


def kernel(features_1, segment_ids):
    f = features_1.reshape(N, F)
    ids = segment_ids.astype(jnp.int32)
    raw = _gavg_pool(f, ids.reshape(NSUB, T))
    return raw.reshape(NUM_GRAPHS, D, 3).transpose(0, 2, 1).reshape(
        NUM_GRAPHS, F)


# speedup vs baseline: 1.4803x; 1.0016x over previous
import jax
import jax.numpy as jnp
from jax import lax
from jax.experimental import pallas as pl
from jax.experimental.pallas import tpu as pltpu
from jax.experimental.pallas import tpu_sc as plsc

N = 100000
D = 128
NUM_GRAPHS = 128
F = 3 * D
NC = 2
NS = 16
L = 16
T = 125
NSUB = N // T
SUB_PER_S = NSUB // NS
ROWS_PER_S = N // NS
CPS = F // NC
GRP = CPS // L


def _sc_body(f_hbm, ids2d_hbm, out_hbm,
             acc, cnt, ids_v, buf_a, buf_b, buf_c, buf_d,
             zbuf, czbuf, cloc, identr, fbuf, cbuf, sem_g, sem_s):
    c = lax.axis_index("c")
    s = lax.axis_index("s")
    col0 = c * CPS
    row0 = s * ROWS_PER_S

    zero = jnp.zeros((L,), jnp.float32)
    for r in range(8):
        for k in range(GRP):
            zbuf[r, pl.ds(k * L, L)] = zero
        czbuf[r, :] = zero
    pltpu.sync_copy(zbuf, acc.at[pl.ds(s * 8, 8)])
    pltpu.sync_copy(czbuf, cnt.at[pl.ds(s * 8, 8)])

    lanes = lax.iota(jnp.int32, L)
    ones = jnp.full((L,), 1.0, jnp.float32)
    for r in range(NUM_GRAPHS):
        cloc[r, :] = zero
    for k in range(NUM_GRAPHS // L):
        identr[0, pl.ds(k * L, L)] = lanes + jnp.full((L,), k * L, jnp.int32)
    pltpu.sync_copy(ids2d_hbm.at[pl.ds(s * SUB_PER_S, SUB_PER_S)], ids_v)
    tail_msk = lanes >= jnp.full((L,), 3, jnp.int32)
    for j in range(SUB_PER_S):
        for k in range(T // L):
            idv = ids_v[j, pl.ds(k * L, L)]
            plsc.addupdate_scatter(cloc, [idv, lanes], ones)
        idv = ids_v[j, pl.ds(T - L, L)]
        plsc.addupdate_scatter(cloc, [idv, lanes], ones, mask=tail_msk)


    bufs = (buf_a, buf_b, buf_c, buf_d)
    nbuf = len(bufs)

    def start_gather(j):
        return pltpu.async_copy(
            f_hbm.at[pl.ds(row0 + j * T, T), pl.ds(col0, CPS)],
            bufs[j % nbuf], sem_g)

    def start_scatter(j):
        return pltpu.async_copy(
            bufs[j % nbuf], acc.at[ids_v.at[j]], sem_s, add=True)

    gds = {}
    sds = {}
    for b in range(2):
        gds[b] = start_gather(b)
    for j in range(SUB_PER_S):
        gds[j].wait()
        sds[j] = start_scatter(j)
        nx = j + 2
        if nx < SUB_PER_S:
            if nx - nbuf >= 0:
                sds[nx - nbuf].wait()
            gds[nx] = start_gather(nx)
    for j in range(max(0, SUB_PER_S - nbuf), SUB_PER_S):
        sds[j].wait()

    pltpu.sync_copy(cloc, cnt.at[identr.at[0]], add=True)

    plsc.subcore_barrier()

    pltpu.sync_copy(acc.at[pl.ds(s * 8, 8)], fbuf)
    pltpu.sync_copy(cnt.at[pl.ds(s * 8, 8)], cbuf)
    for r in range(8):
        total = jnp.sum(cbuf[r, :])
        rec = jnp.full((L,), 1.0, jnp.float32) / jnp.maximum(
            jnp.full((L,), total, jnp.float32),
            jnp.full((L,), 1.0, jnp.float32))
        for k in range(GRP):
            fbuf[r, pl.ds(k * L, L)] = fbuf[r, pl.ds(k * L, L)] * rec
    pltpu.sync_copy(fbuf, out_hbm.at[pl.ds(s * 8, 8), pl.ds(col0, CPS)])


@jax.jit
def _gavg_pool(f, ids2d):
    mesh = plsc.VectorSubcoreMesh(core_axis_name="c", subcore_axis_name="s")
    return pl.kernel(
        _sc_body,
        out_type=jax.ShapeDtypeStruct((NUM_GRAPHS, F), jnp.float32),
        mesh=mesh,
        compiler_params=pltpu.CompilerParams(
            use_tc_tiling_on_sc=False, needs_layout_passes=False),
        scratch_types=[
            pltpu.VMEM_SHARED((NUM_GRAPHS, CPS), jnp.float32),
            pltpu.VMEM_SHARED((NUM_GRAPHS, L), jnp.float32),
            pltpu.VMEM((SUB_PER_S, T), jnp.int32),
            pltpu.VMEM((T, CPS), jnp.float32),
            pltpu.VMEM((T, CPS), jnp.float32),
            pltpu.VMEM((T, CPS), jnp.float32),
            pltpu.VMEM((T, CPS), jnp.float32),
            pltpu.VMEM((8, CPS), jnp.float32),
            pltpu.VMEM((8, L), jnp.float32),
            pltpu.VMEM((NUM_GRAPHS, L), jnp.float32),
            pltpu.VMEM((1, NUM_GRAPHS), jnp.int32),
            pltpu.VMEM((8, CPS), jnp.float32),
            pltpu.VMEM((8, L), jnp.float32),
            pltpu.SemaphoreType.DMA,
            pltpu.SemaphoreType.DMA,
        ],
    )(f, ids2d)


def kernel(features_1, segment_ids):
    f = features_1.reshape(N, F)
    ids = segment_ids.astype(jnp.int32)
    raw = _gavg_pool(f, ids.reshape(NSUB, T))
    return raw.reshape(NUM_GRAPHS, D, 3).transpose(0, 2, 1).reshape(
        NUM_GRAPHS, F)


# speedup vs baseline: 4.6166x; 3.1187x over previous
import jax
import jax.numpy as jnp
from jax import lax
from jax.experimental import pallas as pl
from jax.experimental.pallas import tpu as pltpu
from jax.experimental.pallas import tpu_sc as plsc

N = 100000
D = 128
NUM_GRAPHS = 128
NCOMP = 3
NC = 2
NS = 16
L = 16
T = 125
NSUB = N // T
SUB_PER_S = NSUB // NS
ROWS_PER_S = N // NS
CPS = D // NC
GRP = CPS // L


def _sc_body(f_hbm, ids2d_hbm, out_hbm,
             acc0, acc1, acc2, cnt, ids_v, buf_a, buf_b, buf_c, buf_d,
             zbuf, czbuf, cloc, identr, fbuf, cbuf, sem_g, sem_s):
    c = lax.axis_index("c")
    s = lax.axis_index("s")
    col0 = c * CPS
    row0 = s * ROWS_PER_S
    accs = (acc0, acc1, acc2)

    zero = jnp.zeros((L,), jnp.float32)
    for r in range(8):
        for k in range(GRP):
            zbuf[r, pl.ds(k * L, L)] = zero
        czbuf[r, :] = zero
    for a in accs:
        pltpu.sync_copy(zbuf, a.at[pl.ds(s * 8, 8)])
    pltpu.sync_copy(czbuf, cnt.at[pl.ds(s * 8, 8)])

    lanes = lax.iota(jnp.int32, L)
    ones = jnp.full((L,), 1.0, jnp.float32)
    for r in range(NUM_GRAPHS):
        cloc[r, :] = zero
    for k in range(NUM_GRAPHS // L):
        identr[0, pl.ds(k * L, L)] = lanes + jnp.full((L,), k * L, jnp.int32)
    pltpu.sync_copy(ids2d_hbm.at[pl.ds(s * SUB_PER_S, SUB_PER_S)], ids_v)
    tail_msk = lanes >= jnp.full((L,), 3, jnp.int32)
    for j in range(SUB_PER_S):
        for k in range(T // L):
            idv = ids_v[j, pl.ds(k * L, L)]
            plsc.addupdate_scatter(cloc, [idv, lanes], ones)
        idv = ids_v[j, pl.ds(T - L, L)]
        plsc.addupdate_scatter(cloc, [idv, lanes], ones, mask=tail_msk)

    bufs = (buf_a, buf_b, buf_c, buf_d)
    nbuf = len(bufs)
    nunit = SUB_PER_S * NCOMP

    def start_gather(u):
        j, k = divmod(u, NCOMP)
        return pltpu.async_copy(
            f_hbm.at[pl.ds(k * N + row0 + j * T, T), pl.ds(col0, CPS)],
            bufs[u % nbuf], sem_g)

    def start_scatter(u):
        j, k = divmod(u, NCOMP)
        return pltpu.async_copy(
            bufs[u % nbuf], accs[k].at[ids_v.at[j]], sem_s, add=True)

    gds = {}
    sds = {}
    for b in range(2):
        gds[b] = start_gather(b)
    for u in range(nunit):
        gds[u].wait()
        sds[u] = start_scatter(u)
        nx = u + 2
        if nx < nunit:
            if nx - nbuf >= 0:
                sds[nx - nbuf].wait()
            gds[nx] = start_gather(nx)
    for u in range(nunit - nbuf, nunit):
        sds[u].wait()

    pltpu.sync_copy(cloc, cnt.at[identr.at[0]], add=True)

    plsc.subcore_barrier()

    pltpu.sync_copy(cnt.at[pl.ds(s * 8, 8)], cbuf)
    for k in range(NCOMP):
        pltpu.sync_copy(accs[k].at[pl.ds(s * 8, 8)], fbuf)
        for r in range(8):
            total = jnp.sum(cbuf[r, :])
            rec = jnp.full((L,), 1.0, jnp.float32) / jnp.maximum(
                jnp.full((L,), total, jnp.float32),
                jnp.full((L,), 1.0, jnp.float32))
            for g in range(GRP):
                fbuf[r, pl.ds(g * L, L)] = fbuf[r, pl.ds(g * L, L)] * rec
        pltpu.sync_copy(
            fbuf, out_hbm.at[k, pl.ds(s * 8, 8), pl.ds(col0, CPS)])


@jax.jit
def _gavg_pool(f, ids2d):
    mesh = plsc.VectorSubcoreMesh(core_axis_name="c", subcore_axis_name="s")
    return pl.kernel(
        _sc_body,
        out_type=jax.ShapeDtypeStruct((NCOMP, NUM_GRAPHS, D), jnp.float32),
        mesh=mesh,
        compiler_params=pltpu.CompilerParams(
            use_tc_tiling_on_sc=False, needs_layout_passes=False),
        scratch_types=[
            pltpu.VMEM_SHARED((NUM_GRAPHS, CPS), jnp.float32),
            pltpu.VMEM_SHARED((NUM_GRAPHS, CPS), jnp.float32),
            pltpu.VMEM_SHARED((NUM_GRAPHS, CPS), jnp.float32),
            pltpu.VMEM_SHARED((NUM_GRAPHS, L), jnp.float32),
            pltpu.VMEM((SUB_PER_S, T), jnp.int32),
            pltpu.VMEM((T, CPS), jnp.float32),
            pltpu.VMEM((T, CPS), jnp.float32),
            pltpu.VMEM((T, CPS), jnp.float32),
            pltpu.VMEM((T, CPS), jnp.float32),
            pltpu.VMEM((8, CPS), jnp.float32),
            pltpu.VMEM((8, L), jnp.float32),
            pltpu.VMEM((NUM_GRAPHS, L), jnp.float32),
            pltpu.VMEM((1, NUM_GRAPHS), jnp.int32),
            pltpu.VMEM((8, CPS), jnp.float32),
            pltpu.VMEM((8, L), jnp.float32),
            pltpu.SemaphoreType.DMA,
            pltpu.SemaphoreType.DMA,
        ],
    )(f, ids2d)


def kernel(features_1, segment_ids):
    f = jnp.transpose(features_1, (2, 0, 1)).reshape(NCOMP * N, D)
    ids = segment_ids.astype(jnp.int32)
    raw = _gavg_pool(f, ids.reshape(NSUB, T))
    return jnp.transpose(raw, (1, 0, 2)).reshape(NUM_GRAPHS, NCOMP * D)


# speedup vs baseline: 6.9615x; 1.5079x over previous
import jax
import jax.numpy as jnp
from jax import lax
from jax.experimental import pallas as pl
from jax.experimental.pallas import tpu as pltpu
from jax.experimental.pallas import tpu_sc as plsc

N = 100000
D = 128
NUM_GRAPHS = 128
NCOMP = 3
NC = 2
NS = 16
NW = NC * NS
L = 16
T = 125
NSUB = N // T
ROWS_PER_W = N // NW
SUB_PER_W = ROWS_PER_W // T
CNT_PER_S = NSUB // NS
GRP = D // L
NBUF = 6
INFLIGHT = 3


def _sc_body(f_hbm, ids2d_hbm, out_hbm,
             acc0, acc1, acc2, cnt,
             ids_sc, ids_cn, buf_a, buf_b, buf_c, buf_d, buf_e, buf_f,
             zbuf, czbuf, cloc, identr, fbuf, cbuf, sem_g, sem_s):
    c = lax.axis_index("c")
    s = lax.axis_index("s")
    wid = s * NC + c
    row0 = wid * ROWS_PER_W
    accs = (acc0, acc1, acc2)

    zero = jnp.zeros((L,), jnp.float32)
    for r in range(8):
        for g in range(GRP):
            zbuf[r, pl.ds(g * L, L)] = zero
        czbuf[r, :] = zero
    for a in accs:
        pltpu.sync_copy(zbuf, a.at[pl.ds(s * 8, 8)])
    pltpu.sync_copy(czbuf, cnt.at[pl.ds(s * 8, 8)])

    lanes = lax.iota(jnp.int32, L)
    ones = jnp.full((L,), 1.0, jnp.float32)
    for r in range(NUM_GRAPHS):
        cloc[r, :] = zero
    for g in range(NUM_GRAPHS // L):
        identr[0, pl.ds(g * L, L)] = lanes + jnp.full((L,), g * L, jnp.int32)
    pltpu.sync_copy(ids2d_hbm.at[pl.ds(s * CNT_PER_S, CNT_PER_S)], ids_cn)
    tail_msk = lanes >= jnp.full((L,), 3, jnp.int32)
    for j in range(CNT_PER_S):
        for g in range(T // L):
            idv = ids_cn[j, pl.ds(g * L, L)]
            plsc.addupdate_scatter(cloc, [idv, lanes], ones)
        idv = ids_cn[j, pl.ds(T - L, L)]
        plsc.addupdate_scatter(cloc, [idv, lanes], ones, mask=tail_msk)

    pltpu.sync_copy(ids2d_hbm.at[pl.ds(wid * SUB_PER_W, SUB_PER_W)], ids_sc)
    bufs = (buf_a, buf_b, buf_c, buf_d, buf_e, buf_f)
    nunit = NCOMP * SUB_PER_W

    def start_gather(u):
        k, j = divmod(u, SUB_PER_W)
        return pltpu.async_copy(
            f_hbm.at[pl.ds(k * N + row0 + j * T, T), :],
            bufs[u % NBUF], sem_g)

    def start_scatter(u):
        k, j = divmod(u, SUB_PER_W)
        return pltpu.async_copy(
            bufs[u % NBUF], accs[k].at[ids_sc.at[j]], sem_s, add=True)

    gds = {}
    sds = {}
    for b in range(INFLIGHT):
        gds[b] = start_gather(b)
    for u in range(nunit):
        gds[u].wait()
        sds[u] = start_scatter(u)
        nx = u + INFLIGHT
        if nx < nunit:
            if nx - NBUF >= 0:
                sds[nx - NBUF].wait()
            gds[nx] = start_gather(nx)
    for u in range(nunit - NBUF, nunit):
        sds[u].wait()

    pltpu.sync_copy(cloc, cnt.at[identr.at[0]], add=True)

    plsc.subcore_barrier()

    pltpu.sync_copy(cnt.at[pl.ds(s * 8, 8)], cbuf)
    for k in range(NCOMP):
        pltpu.sync_copy(accs[k].at[pl.ds(s * 8, 8)], fbuf)
        for r in range(8):
            total = jnp.sum(cbuf[r, :])
            rec = jnp.full((L,), 1.0, jnp.float32) / jnp.maximum(
                jnp.full((L,), total, jnp.float32),
                jnp.full((L,), 1.0, jnp.float32))
            for g in range(GRP):
                fbuf[r, pl.ds(g * L, L)] = fbuf[r, pl.ds(g * L, L)] * rec
        pltpu.sync_copy(
            fbuf, out_hbm.at[c, k, pl.ds(s * 8, 8), :])


@jax.jit
def _gavg_pool(f, ids2d):
    mesh = plsc.VectorSubcoreMesh(core_axis_name="c", subcore_axis_name="s")
    return pl.kernel(
        _sc_body,
        out_type=jax.ShapeDtypeStruct((NC, NCOMP, NUM_GRAPHS, D),
                                      jnp.float32),
        mesh=mesh,
        compiler_params=pltpu.CompilerParams(
            use_tc_tiling_on_sc=False, needs_layout_passes=False),
        scratch_types=[
            pltpu.VMEM_SHARED((NUM_GRAPHS, D), jnp.float32),
            pltpu.VMEM_SHARED((NUM_GRAPHS, D), jnp.float32),
            pltpu.VMEM_SHARED((NUM_GRAPHS, D), jnp.float32),
            pltpu.VMEM_SHARED((NUM_GRAPHS, L), jnp.float32),
            pltpu.VMEM((SUB_PER_W, T), jnp.int32),
            pltpu.VMEM((CNT_PER_S, T), jnp.int32),
            pltpu.VMEM((T, D), jnp.float32),
            pltpu.VMEM((T, D), jnp.float32),
            pltpu.VMEM((T, D), jnp.float32),
            pltpu.VMEM((T, D), jnp.float32),
            pltpu.VMEM((T, D), jnp.float32),
            pltpu.VMEM((T, D), jnp.float32),
            pltpu.VMEM((8, D), jnp.float32),
            pltpu.VMEM((8, L), jnp.float32),
            pltpu.VMEM((NUM_GRAPHS, L), jnp.float32),
            pltpu.VMEM((1, NUM_GRAPHS), jnp.int32),
            pltpu.VMEM((8, D), jnp.float32),
            pltpu.VMEM((8, L), jnp.float32),
            pltpu.SemaphoreType.DMA,
            pltpu.SemaphoreType.DMA,
        ],
    )(f, ids2d)


def kernel(features_1, segment_ids):
    f = jnp.transpose(features_1, (2, 0, 1)).reshape(NCOMP * N, D)
    ids = segment_ids.astype(jnp.int32)
    raw = _gavg_pool(f, ids.reshape(NSUB, T))
    comb = raw[0] + raw[1]
    return jnp.transpose(comb, (1, 0, 2)).reshape(NUM_GRAPHS, NCOMP * D)
